# baseline (speedup 1.0000x reference)
import functools

import jax
import jax.numpy as jnp
from jax import lax
from jax.experimental import pallas as pl
from jax.experimental.pallas import tpu as pltpu
from jax.experimental.pallas import tpu_sc as plsc

POSE_NUM = 1_000_000
EMBED_DIM = 6
BATCH = 16384

_NUM_CORES = 2
_NUM_SUBCORES = 16
_NW = _NUM_CORES * _NUM_SUBCORES
_BPW = BATCH // _NW
_K = 32
_NCHUNK = _BPW // _K
_NBUF = 2
_L = 16

_TILE_W = 128
_LAST_TILE = (POSE_NUM - 1) // _TILE_W
_TAIL_START = _LAST_TILE * _TILE_W
_TAIL_W = POSE_NUM - _TAIL_START
_MAX_FULL_OFF = (_LAST_TILE - 1) * _TILE_W

_mesh = plsc.VectorSubcoreMesh(core_axis_name="c", subcore_axis_name="s")


@functools.partial(
    pl.kernel,
    mesh=_mesh,
    compiler_params=pltpu.CompilerParams(needs_layout_passes=False),
    out_type=jax.ShapeDtypeStruct((EMBED_DIM, BATCH), jnp.float32),
    scratch_types=[
        pltpu.VMEM((_BPW,), jnp.int32),
        pltpu.VMEM((_NBUF * _K * 8, _TILE_W), jnp.float32),
        pltpu.VMEM((8, _TILE_W), jnp.float32),
        pltpu.VMEM((EMBED_DIM, _BPW), jnp.float32),
        pltpu.SemaphoreType.DMA,
    ],
)
def _sc_gather(idx_hbm, table_hbm, tailp_hbm, out_hbm, idx_v, blk_v, tail_v, cols_v, sem):
    wid = lax.axis_index("s") * _NUM_CORES + lax.axis_index("c")
    base = wid * _BPW
    pltpu.sync_copy(idx_hbm.at[pl.ds(base, _BPW)], idx_v)
    pltpu.sync_copy(tailp_hbm, tail_v.at[pl.ds(0, EMBED_DIM), :])

    def _fire_chunk(k):
        b = k % _NBUF
        cps = []
        for h in range(_K // _L):
            v = idx_v[pl.ds(k * _K + h * _L, _L)]
            for lane in range(_L):
                r = v[lane]
                t = r >> 7
                t = jnp.minimum(t, _LAST_TILE - 1)
                off = pl.multiple_of(t * _TILE_W, _TILE_W)
                cps.append(
                    pltpu.async_copy(
                        table_hbm.at[:, pl.ds(off, _TILE_W)],
                        blk_v.at[pl.ds((b * _K + h * _L + lane) * 8, EMBED_DIM), :],
                        sem,
                    )
                )
        return cps

    def _extract_chunk(k):
        b = k % _NBUF
        for h in range(_K // _L):
            v = idx_v[pl.ds(k * _K + h * _L, _L)]
            lane = v & (_TILE_W - 1)
            jvec = lax.iota(jnp.int32, _L) + (b * _K + h * _L)
            is_tail = v >= _TAIL_START
            tail_col = jnp.minimum(v - _TAIL_START, _TAIL_W - 1)
            tail_col = jnp.where(is_tail, tail_col, 0)
            for c in range(EMBED_DIM):
                cvec = jnp.full((_L,), c, jnp.int32)
                main = plsc.load_gather(blk_v, [jvec * 8 + c, lane])
                tail = plsc.load_gather(tail_v, [cvec, tail_col])
                cols_v[c, pl.ds(k * _K + h * _L, _L)] = jnp.where(
                    is_tail, tail, main
                )

    pending = _fire_chunk(0)
    for k in range(_NCHUNK):
        nxt = _fire_chunk(k + 1) if k + 1 < _NCHUNK else []
        for cp in pending:
            cp.wait()
        _extract_chunk(k)
        pending = nxt

    for c in range(EMBED_DIM):
        pltpu.sync_copy(
            cols_v.at[pl.ds(c, 1), :],
            out_hbm.at[pl.ds(c, 1), pl.ds(base, _BPW)],
        )


def kernel(indices, table):
    table_t = table.T
    tail_pad = jnp.pad(
        table_t[:, _TAIL_START:], ((0, 0), (0, _TILE_W - _TAIL_W))
    )
    out_t = _sc_gather(indices.astype(jnp.int32), table_t, tail_pad)
    return out_t.T


# speedup vs baseline: 1.0207x; 1.0207x over previous
import functools

import jax
import jax.numpy as jnp
from jax import lax
from jax.experimental import pallas as pl
from jax.experimental.pallas import tpu as pltpu
from jax.experimental.pallas import tpu_sc as plsc

POSE_NUM = 1_000_000
EMBED_DIM = 6
BATCH = 16384

_NUM_CORES = 2
_NUM_SUBCORES = 16
_NW = _NUM_CORES * _NUM_SUBCORES
_BPW = BATCH // _NW
_K = 32
_NCHUNK = _BPW // _K
_NBUF = 2
_L = 16

_TILE_W = 128
_LAST_TILE = (POSE_NUM - 1) // _TILE_W
_TAIL_START = _LAST_TILE * _TILE_W
_TAIL_W = POSE_NUM - _TAIL_START
_MAX_FULL_OFF = (_LAST_TILE - 1) * _TILE_W

_mesh = plsc.VectorSubcoreMesh(core_axis_name="c", subcore_axis_name="s")


@functools.partial(
    pl.kernel,
    mesh=_mesh,
    compiler_params=pltpu.CompilerParams(needs_layout_passes=False),
    out_type=jax.ShapeDtypeStruct((EMBED_DIM, BATCH), jnp.float32),
    scratch_types=[
        pltpu.VMEM((_BPW,), jnp.int32),
        pltpu.VMEM((_NBUF * _K * 8, _TILE_W), jnp.float32),
        pltpu.VMEM((8, _TILE_W), jnp.float32),
        pltpu.VMEM((EMBED_DIM, _BPW), jnp.float32),
        pltpu.SemaphoreType.DMA,
    ],
)
def _sc_gather(idx_hbm, table_hbm, tailp_hbm, out_hbm, idx_v, blk_v, tail_v, cols_v, sem):
    wid = lax.axis_index("s") * _NUM_CORES + lax.axis_index("c")
    base = wid * _BPW
    pltpu.sync_copy(idx_hbm.at[pl.ds(base, _BPW)], idx_v)
    tail_cp = pltpu.async_copy(
        tailp_hbm, tail_v.at[pl.ds(0, EMBED_DIM), :], sem
    )

    def _fire_chunk(k):
        b = k % _NBUF
        cps = []
        for h in range(_K // _L):
            v = idx_v[pl.ds(k * _K + h * _L, _L)]
            for lane in range(_L):
                r = v[lane]
                t = r >> 7
                t = jnp.minimum(t, _LAST_TILE - 1)
                off = pl.multiple_of(t * _TILE_W, _TILE_W)
                cps.append(
                    pltpu.async_copy(
                        table_hbm.at[:, pl.ds(off, _TILE_W)],
                        blk_v.at[pl.ds((b * _K + h * _L + lane) * 8, EMBED_DIM), :],
                        sem,
                    )
                )
        return cps

    def _extract_chunk(k):
        b = k % _NBUF
        for h in range(_K // _L):
            v = idx_v[pl.ds(k * _K + h * _L, _L)]
            lane = v & (_TILE_W - 1)
            jvec = lax.iota(jnp.int32, _L) + (b * _K + h * _L)
            is_tail = v >= _TAIL_START
            tail_col = jnp.minimum(v - _TAIL_START, _TAIL_W - 1)
            tail_col = jnp.where(is_tail, tail_col, 0)
            for c in range(EMBED_DIM):
                cvec = jnp.full((_L,), c, jnp.int32)
                main = plsc.load_gather(blk_v, [jvec * 8 + c, lane])
                tail = plsc.load_gather(tail_v, [cvec, tail_col])
                cols_v[c, pl.ds(k * _K + h * _L, _L)] = jnp.where(
                    is_tail, tail, main
                )

    pending = _fire_chunk(0)
    tail_cp.wait()
    for k in range(_NCHUNK):
        nxt = _fire_chunk(k + 1) if k + 1 < _NCHUNK else []
        for cp in pending:
            cp.wait()
        _extract_chunk(k)
        pending = nxt

    out_cps = [
        pltpu.async_copy(
            cols_v.at[pl.ds(c, 1), :],
            out_hbm.at[pl.ds(c, 1), pl.ds(base, _BPW)],
            sem,
        )
        for c in range(EMBED_DIM)
    ]
    for cp in out_cps:
        cp.wait()


def kernel(indices, table):
    table_t = table.T
    tail_pad = jnp.pad(
        table_t[:, _TAIL_START:], ((0, 0), (0, _TILE_W - _TAIL_W))
    )
    out_t = _sc_gather(indices.astype(jnp.int32), table_t, tail_pad)
    return out_t.T
